# baseline (speedup 1.0000x reference)
import dataclasses
import functools

import jax
import jax.numpy as jnp
from jax import lax
from jax.experimental import pallas as pl
from jax.experimental.pallas import tpu as pltpu
from jax.experimental.pallas import tpu_sc as plsc


def _sc_aggregate(x, ei4, zeros_feat, zeros_deg, Np, n_per_tile):
    N, D = x.shape
    mesh = plsc.VectorSubcoreMesh(core_axis_name="c", subcore_axis_name="s")
    NC, NS = mesh.num_cores, mesh.num_subcores
    NW = NC * NS
    _, nch, _, C = ei4.shape
    assert nch >= 5
    main = ((nch - 1) // 4) * 4

    @functools.partial(
        pl.kernel,
        out_type=(
            jax.ShapeDtypeStruct((NC, Np, D), jnp.float32),
            jax.ShapeDtypeStruct((NW, 1, Np), jnp.float32),
        ),
        mesh=mesh,
        scratch_types=[
            pltpu.VMEM((2, C), jnp.int32),
            pltpu.VMEM((2, C), jnp.int32),
            pltpu.VMEM((2, C), jnp.int32),
            pltpu.VMEM((2, C), jnp.int32),
            pltpu.VMEM((C, D), jnp.float32),
            pltpu.VMEM((C, D), jnp.float32),
            pltpu.VMEM((1, Np), jnp.float32),
            pltpu.VMEM_SHARED((Np, D), jnp.float32),
            pltpu.SemaphoreType.DMA,
            pltpu.SemaphoreType.DMA,
            pltpu.SemaphoreType.DMA,
            pltpu.SemaphoreType.DMA,
            pltpu.SemaphoreType.DMA,
            pltpu.SemaphoreType.DMA,
            pltpu.SemaphoreType.DMA,
            pltpu.SemaphoreType.DMA,
        ],
        compiler_params=dataclasses.replace(pltpu.CompilerParams(),
                                            needs_layout_passes=False),
    )
    def agg_kernel(x_hbm, ei_hbm, zf_hbm, zd_hbm, agg_out, deg_out,
                   idx0, idx1, idx2, idx3, rows0, rows1, deg_v, agg_sh,
                   sem_i0, sem_i1, sem_i2, sem_i3,
                   sem_g0, sem_g1, sem_s0, sem_s1):
        core = lax.axis_index("c")
        sub = lax.axis_index("s")
        wid = core * NS + sub
        base_n = sub * n_per_tile
        idxb = (idx0, idx1, idx2, idx3)
        sem_i = (sem_i0, sem_i1, sem_i2, sem_i3)
        rows = (rows0, rows1)
        sem_g = (sem_g0, sem_g1)
        sem_s = (sem_s0, sem_s1)

        def idx_copy(c, b4):
            return pltpu.make_async_copy(ei_hbm.at[wid, c], idxb[b4],
                                         sem_i[b4])

        def gather(c, rb, b4):
            return pltpu.make_async_copy(x_hbm.at[idxb[b4].at[0]], rows[rb],
                                         sem_g[rb])

        def scatter_wait(rb, b4):
            pltpu.make_async_copy(rows[rb], agg_sh.at[idxb[b4].at[1]],
                                  sem_s[rb]).wait()

        idx_copy(0, 0).start()
        idx_copy(1, 1).start()
        idx_copy(2, 2).start()
        pltpu.sync_copy(zf_hbm, agg_sh.at[pl.ds(base_n, n_per_tile)])
        pltpu.sync_copy(zd_hbm, deg_v)
        idx_copy(0, 0).wait()
        gather(0, 0, 0).start()
        plsc.subcore_barrier()

        zero16 = jnp.zeros((16,), jnp.int32)
        one16 = jnp.ones((16,), jnp.float32)

        def hist(b4):
            for j in range(C // 16):
                idx = idxb[b4][1, pl.ds(j * 16, 16)]
                plsc.addupdate_scatter(deg_v, [zero16, idx], one16)

        @pl.loop(0, main, step=4)
        def _(i):
            for b in range(4):
                c = i + b
                rb = b & 1
                b4 = b & 3
                gather(c, rb, b4).wait()

                @pl.when(c >= 1)
                def _():
                    scatter_wait(rb ^ 1, (b4 - 1) % 4)

                idx_copy(c + 1, (b4 + 1) % 4).wait()
                gather(c + 1, rb ^ 1, (b4 + 1) % 4).start()

                @pl.when(c + 3 < nch)
                def _():
                    idx_copy(c + 3, (b4 + 3) % 4).start()

                pltpu.async_copy(rows[rb], agg_sh.at[idxb[b4].at[1]],
                                 sem_s[rb], add=True)
                hist(b4)

        for c in range(main, nch):
            rb, b4 = c & 1, c % 4
            gather(c, rb, b4).wait()
            if c >= 1:
                scatter_wait((c - 1) & 1, (c - 1) % 4)
            if c + 1 < nch:
                idx_copy(c + 1, (c + 1) % 4).wait()
                gather(c + 1, (c + 1) & 1, (c + 1) % 4).start()
            if c + 3 < nch:
                idx_copy(c + 3, (c + 3) % 4).start()
            pltpu.async_copy(rows[rb], agg_sh.at[idxb[b4].at[1]],
                             sem_s[rb], add=True)
            hist(b4)
        scatter_wait((nch - 1) & 1, (nch - 1) % 4)

        plsc.subcore_barrier()
        pltpu.sync_copy(agg_sh.at[pl.ds(base_n, n_per_tile)],
                        agg_out.at[core, pl.ds(base_n, n_per_tile)])
        pltpu.sync_copy(deg_v, deg_out.at[wid])

    return agg_kernel(x, ei4, zeros_feat, zeros_deg)


def _tc_combine_kernel(agg_ref, deg_ref, x_ref, wl_ref, bl_ref, wr_ref, o_ref):
    agg = agg_ref[0] + agg_ref[1]
    deg = jnp.sum(deg_ref[:, 0, :], axis=0, keepdims=True)
    recip = 1.0 / jnp.maximum(deg, 1.0)
    ones_row = jnp.ones((1, agg.shape[1]), jnp.float32)
    recip_col = lax.dot_general(recip, ones_row, (((0,), (0,)), ((), ())),
                                precision=lax.Precision.HIGHEST,
                                preferred_element_type=jnp.float32)
    mean = agg * recip_col
    out = (jnp.dot(mean, wl_ref[...], preferred_element_type=jnp.float32)
           + bl_ref[...]
           + jnp.dot(x_ref[...], wr_ref[...], preferred_element_type=jnp.float32))
    o_ref[...] = jnp.maximum(out, 0.0)


def _tc_combine(agg_p, deg_p, x_pad, W_l, b_l, W_r):
    Np, D = x_pad.shape
    NW = deg_p.shape[0]
    b2 = b_l.reshape(1, D)
    return pl.pallas_call(
        _tc_combine_kernel,
        grid=(1,),
        in_specs=[
            pl.BlockSpec((2, Np, D), lambda i: (0, 0, 0)),
            pl.BlockSpec((NW, 1, Np), lambda i: (0, 0, 0)),
            pl.BlockSpec((Np, D), lambda i: (0, 0)),
            pl.BlockSpec((D, D), lambda i: (0, 0)),
            pl.BlockSpec((1, D), lambda i: (0, 0)),
            pl.BlockSpec((D, D), lambda i: (0, 0)),
        ],
        out_specs=pl.BlockSpec((Np, D), lambda i: (0, 0)),
        out_shape=jax.ShapeDtypeStruct((Np, D), jnp.float32),
    )(agg_p, deg_p, x_pad, W_l, b2, W_r)


def kernel(x, edge_index, W_l, b_l, W_r):
    N, D = x.shape
    NS, NW = 16, 32
    n_per_tile = (N + NS * 8 - 1) // (NS * 8) * 8
    Np = n_per_tile * NS
    E = edge_index.shape[1]
    epw = E // NW
    assert epw * NW == E and N < Np
    C = 112
    nch = (epw + C - 1) // C
    pad = nch * C - epw
    ei3 = edge_index.reshape(2, NW, epw)
    srcp = jnp.pad(ei3[0], ((0, 0), (0, pad)))
    dpad = N + jnp.arange(pad, dtype=jnp.int32) % (Np - N)
    dstp = jnp.concatenate(
        [ei3[1], jnp.broadcast_to(dpad, (NW, pad))], axis=1)
    ei4 = jnp.stack([srcp, dstp]).reshape(2, NW, nch, C).transpose(1, 2, 0, 3)
    zeros_feat = jnp.zeros((n_per_tile, D), jnp.float32)
    zeros_deg = jnp.zeros((1, Np), jnp.float32)
    agg_p, deg_p = _sc_aggregate(x, ei4, zeros_feat, zeros_deg,
                                 Np, n_per_tile)
    x_pad = jnp.pad(x, ((0, Np - N), (0, 0)))
    return _tc_combine(agg_p, deg_p, x_pad, W_l, b_l, W_r)[:N]


# speedup vs baseline: 1.3028x; 1.3028x over previous
import dataclasses
import functools

import jax
import jax.numpy as jnp
from jax import lax
from jax.experimental import pallas as pl
from jax.experimental.pallas import tpu as pltpu
from jax.experimental.pallas import tpu_sc as plsc


def _sc_aggregate(x, ei4, zeros_feat, zeros_deg, Np, n_per_tile):
    N, D = x.shape
    mesh = plsc.VectorSubcoreMesh(core_axis_name="c", subcore_axis_name="s")
    NC, NS = mesh.num_cores, mesh.num_subcores
    NW = NC * NS
    _, nch, _, C = ei4.shape
    assert nch >= 5
    main = ((nch - 1) // 4) * 4

    @functools.partial(
        pl.kernel,
        out_type=(
            jax.ShapeDtypeStruct((NC, Np, D), jnp.float32),
            jax.ShapeDtypeStruct((NW, 1, Np), jnp.float32),
        ),
        mesh=mesh,
        scratch_types=[
            pltpu.VMEM((2, C), jnp.int32),
            pltpu.VMEM((2, C), jnp.int32),
            pltpu.VMEM((2, C), jnp.int32),
            pltpu.VMEM((2, C), jnp.int32),
            pltpu.VMEM((C, D), jnp.float32),
            pltpu.VMEM((C, D), jnp.float32),
            pltpu.VMEM((1, Np), jnp.float32),
            pltpu.VMEM_SHARED((Np, D), jnp.float32),
            pltpu.SemaphoreType.DMA,
            pltpu.SemaphoreType.DMA,
            pltpu.SemaphoreType.DMA,
            pltpu.SemaphoreType.DMA,
            pltpu.SemaphoreType.DMA,
            pltpu.SemaphoreType.DMA,
            pltpu.SemaphoreType.DMA,
            pltpu.SemaphoreType.DMA,
        ],
        compiler_params=dataclasses.replace(pltpu.CompilerParams(),
                                            needs_layout_passes=False),
    )
    def agg_kernel(x_hbm, ei_hbm, zf_hbm, zd_hbm, agg_out, deg_out,
                   idx0, idx1, idx2, idx3, rows0, rows1, deg_v, agg_sh,
                   sem_i0, sem_i1, sem_i2, sem_i3,
                   sem_g0, sem_g1, sem_s0, sem_s1):
        core = lax.axis_index("c")
        sub = lax.axis_index("s")
        wid = core * NS + sub
        base_n = sub * n_per_tile
        idxb = (idx0, idx1, idx2, idx3)
        sem_i = (sem_i0, sem_i1, sem_i2, sem_i3)
        rows = (rows0, rows1)
        sem_g = (sem_g0, sem_g1)
        sem_s = (sem_s0, sem_s1)

        def idx_copy(c, b4):
            return pltpu.make_async_copy(ei_hbm.at[wid, c], idxb[b4],
                                         sem_i[b4])

        def gather(c, rb, b4):
            return pltpu.make_async_copy(x_hbm.at[idxb[b4].at[0]], rows[rb],
                                         sem_g[rb])

        def scatter_wait(rb, b4):
            pltpu.make_async_copy(rows[rb], agg_sh.at[idxb[b4].at[1]],
                                  sem_s[rb]).wait()

        idx_copy(0, 0).start()
        idx_copy(1, 1).start()
        idx_copy(2, 2).start()
        pltpu.sync_copy(zf_hbm, agg_sh.at[pl.ds(base_n, n_per_tile)])
        pltpu.sync_copy(zd_hbm, deg_v)
        idx_copy(0, 0).wait()
        gather(0, 0, 0).start()
        plsc.subcore_barrier()

        zero16 = jnp.zeros((16,), jnp.int32)
        one16 = jnp.ones((16,), jnp.float32)

        def hist(b4):
            for j in range(C // 16):
                idx = idxb[b4][1, pl.ds(j * 16, 16)]
                plsc.addupdate_scatter(deg_v, [zero16, idx], one16)

        @pl.loop(0, main, step=4)
        def _(i):
            for b in range(4):
                c = i + b
                rb = b & 1
                b4 = b & 3
                gather(c, rb, b4).wait()

                @pl.when(c >= 1)
                def _():
                    scatter_wait(rb ^ 1, (b4 - 1) % 4)

                idx_copy(c + 1, (b4 + 1) % 4).wait()
                gather(c + 1, rb ^ 1, (b4 + 1) % 4).start()

                @pl.when(c + 3 < nch)
                def _():
                    idx_copy(c + 3, (b4 + 3) % 4).start()

                pltpu.async_copy(rows[rb], agg_sh.at[idxb[b4].at[1]],
                                 sem_s[rb], add=True)
                hist(b4)

        for c in range(main, nch):
            rb, b4 = c & 1, c % 4
            gather(c, rb, b4).wait()
            if c >= 1:
                scatter_wait((c - 1) & 1, (c - 1) % 4)
            if c + 1 < nch:
                idx_copy(c + 1, (c + 1) % 4).wait()
                gather(c + 1, (c + 1) & 1, (c + 1) % 4).start()
            if c + 3 < nch:
                idx_copy(c + 3, (c + 3) % 4).start()
            pltpu.async_copy(rows[rb], agg_sh.at[idxb[b4].at[1]],
                             sem_s[rb], add=True)
            hist(b4)
        scatter_wait((nch - 1) & 1, (nch - 1) % 4)

        plsc.subcore_barrier()
        pltpu.sync_copy(agg_sh.at[pl.ds(base_n, n_per_tile)],
                        agg_out.at[core, pl.ds(base_n, n_per_tile)])
        pltpu.sync_copy(deg_v, deg_out.at[wid])

    return agg_kernel(x, ei4, zeros_feat, zeros_deg)


def _tc_combine_kernel(agg_ref, deg_ref, x_ref, wl_ref, bl_ref, wr_ref, o_ref):
    agg = agg_ref[0] + agg_ref[1]
    deg = jnp.sum(deg_ref[:, 0, :], axis=0, keepdims=True)
    recip = 1.0 / jnp.maximum(deg, 1.0)
    ones_row = jnp.ones((1, agg.shape[1]), jnp.float32)
    recip_col = lax.dot_general(recip, ones_row, (((0,), (0,)), ((), ())),
                                precision=lax.Precision.HIGHEST,
                                preferred_element_type=jnp.float32)
    mean = agg * recip_col
    out = (jnp.dot(mean, wl_ref[...], preferred_element_type=jnp.float32)
           + bl_ref[...]
           + jnp.dot(x_ref[...], wr_ref[...], preferred_element_type=jnp.float32))
    o_ref[...] = jnp.maximum(out, 0.0)


def _tc_combine(agg_p, deg_p, x_pad, W_l, b_l, W_r):
    Np, D = x_pad.shape
    NW = deg_p.shape[0]
    b2 = b_l.reshape(1, D)
    return pl.pallas_call(
        _tc_combine_kernel,
        grid=(1,),
        in_specs=[
            pl.BlockSpec((2, Np, D), lambda i: (0, 0, 0)),
            pl.BlockSpec((NW, 1, Np), lambda i: (0, 0, 0)),
            pl.BlockSpec((Np, D), lambda i: (0, 0)),
            pl.BlockSpec((D, D), lambda i: (0, 0)),
            pl.BlockSpec((1, D), lambda i: (0, 0)),
            pl.BlockSpec((D, D), lambda i: (0, 0)),
        ],
        out_specs=pl.BlockSpec((Np, D), lambda i: (0, 0)),
        out_shape=jax.ShapeDtypeStruct((Np, D), jnp.float32),
    )(agg_p, deg_p, x_pad, W_l, b2, W_r)


def kernel(x, edge_index, W_l, b_l, W_r):
    N, D = x.shape
    NS, NW = 16, 32
    n_per_tile = (N + NS * 8 - 1) // (NS * 8) * 8
    Np = n_per_tile * NS
    E = edge_index.shape[1]
    epw = E // NW
    assert epw * NW == E and N < Np
    C = 80
    nch = (epw + C - 1) // C
    pad = nch * C - epw
    ei3 = edge_index.reshape(2, NW, epw)
    srcp = jnp.pad(ei3[0], ((0, 0), (0, pad)))
    dpad = N + jnp.arange(pad, dtype=jnp.int32) % (Np - N)
    dstp = jnp.concatenate(
        [ei3[1], jnp.broadcast_to(dpad, (NW, pad))], axis=1)
    ei4 = jnp.stack([srcp, dstp]).reshape(2, NW, nch, C).transpose(1, 2, 0, 3)
    zeros_feat = jnp.zeros((n_per_tile, D), jnp.float32)
    zeros_deg = jnp.zeros((1, Np), jnp.float32)
    agg_p, deg_p = _sc_aggregate(x, ei4, zeros_feat, zeros_deg,
                                 Np, n_per_tile)
    x_pad = jnp.pad(x, ((0, Np - N), (0, 0)))
    return _tc_combine(agg_p, deg_p, x_pad, W_l, b_l, W_r)[:N]


# speedup vs baseline: 1.4539x; 1.1160x over previous
import dataclasses
import functools

import jax
import jax.numpy as jnp
from jax import lax
from jax.experimental import pallas as pl
from jax.experimental.pallas import tpu as pltpu
from jax.experimental.pallas import tpu_sc as plsc


def _sc_aggregate(x, ei4, zeros_feat, zeros_deg, Np, n_per_tile):
    N, D = x.shape
    mesh = plsc.VectorSubcoreMesh(core_axis_name="c", subcore_axis_name="s")
    NC, NS = mesh.num_cores, mesh.num_subcores
    NW = NC * NS
    _, nch, _, C = ei4.shape
    assert nch >= 5
    main = ((nch - 1) // 4) * 4

    @functools.partial(
        pl.kernel,
        out_type=(
            jax.ShapeDtypeStruct((NC, Np, D), jnp.float32),
            jax.ShapeDtypeStruct((NW, 1, Np), jnp.float32),
        ),
        mesh=mesh,
        scratch_types=[
            pltpu.VMEM((2, C), jnp.int32),
            pltpu.VMEM((2, C), jnp.int32),
            pltpu.VMEM((2, C), jnp.int32),
            pltpu.VMEM((2, C), jnp.int32),
            pltpu.VMEM((C, D), jnp.float32),
            pltpu.VMEM((C, D), jnp.float32),
            pltpu.VMEM((1, Np), jnp.float32),
            pltpu.VMEM_SHARED((Np, D), jnp.float32),
            pltpu.SemaphoreType.DMA,
            pltpu.SemaphoreType.DMA,
            pltpu.SemaphoreType.DMA,
            pltpu.SemaphoreType.DMA,
            pltpu.SemaphoreType.DMA,
            pltpu.SemaphoreType.DMA,
            pltpu.SemaphoreType.DMA,
            pltpu.SemaphoreType.DMA,
        ],
        compiler_params=dataclasses.replace(pltpu.CompilerParams(),
                                            needs_layout_passes=False),
    )
    def agg_kernel(x_hbm, ei_hbm, zf_hbm, zd_hbm, agg_out, deg_out,
                   idx0, idx1, idx2, idx3, rows0, rows1, deg_v, agg_sh,
                   sem_i0, sem_i1, sem_i2, sem_i3,
                   sem_g0, sem_g1, sem_s0, sem_s1):
        core = lax.axis_index("c")
        sub = lax.axis_index("s")
        wid = core * NS + sub
        base_n = sub * n_per_tile
        idxb = (idx0, idx1, idx2, idx3)
        sem_i = (sem_i0, sem_i1, sem_i2, sem_i3)
        rows = (rows0, rows1)
        sem_g = (sem_g0, sem_g1)
        sem_s = (sem_s0, sem_s1)

        def idx_copy(c, b4):
            return pltpu.make_async_copy(ei_hbm.at[wid, c], idxb[b4],
                                         sem_i[b4])

        def gather(c, rb, b4):
            return pltpu.make_async_copy(x_hbm.at[idxb[b4].at[0]], rows[rb],
                                         sem_g[rb])

        def scatter_wait(rb, b4):
            pltpu.make_async_copy(rows[rb], agg_sh.at[idxb[b4].at[1]],
                                  sem_s[rb]).wait()

        idx_copy(0, 0).start()
        idx_copy(1, 1).start()
        idx_copy(2, 2).start()
        pltpu.sync_copy(zf_hbm, agg_sh.at[pl.ds(base_n, n_per_tile)])
        pltpu.sync_copy(zd_hbm, deg_v)
        idx_copy(0, 0).wait()
        gather(0, 0, 0).start()
        plsc.subcore_barrier()

        zero16 = jnp.zeros((16,), jnp.int32)
        one16 = jnp.ones((16,), jnp.float32)

        def hist(b4):
            for j in range(C // 16):
                idx = idxb[b4][1, pl.ds(j * 16, 16)]
                plsc.addupdate_scatter(deg_v, [zero16, idx], one16)

        @pl.loop(0, main, step=4)
        def _(i):
            for b in range(4):
                c = i + b
                rb = b & 1
                b4 = b & 3
                gather(c, rb, b4).wait()

                @pl.when(c >= 1)
                def _():
                    scatter_wait(rb ^ 1, (b4 - 1) % 4)

                idx_copy(c + 1, (b4 + 1) % 4).wait()
                gather(c + 1, rb ^ 1, (b4 + 1) % 4).start()

                @pl.when(c + 3 < nch)
                def _():
                    idx_copy(c + 3, (b4 + 3) % 4).start()

                pltpu.async_copy(rows[rb], agg_sh.at[idxb[b4].at[1]],
                                 sem_s[rb], add=True)
                hist(b4)

        for c in range(main, nch):
            rb, b4 = c & 1, c % 4
            gather(c, rb, b4).wait()
            if c >= 1:
                scatter_wait((c - 1) & 1, (c - 1) % 4)
            if c + 1 < nch:
                idx_copy(c + 1, (c + 1) % 4).wait()
                gather(c + 1, (c + 1) & 1, (c + 1) % 4).start()
            if c + 3 < nch:
                idx_copy(c + 3, (c + 3) % 4).start()
            pltpu.async_copy(rows[rb], agg_sh.at[idxb[b4].at[1]],
                             sem_s[rb], add=True)
            hist(b4)
        scatter_wait((nch - 1) & 1, (nch - 1) % 4)

        plsc.subcore_barrier()
        pltpu.sync_copy(agg_sh.at[pl.ds(base_n, n_per_tile)],
                        agg_out.at[core, pl.ds(base_n, n_per_tile)])
        pltpu.sync_copy(deg_v, deg_out.at[wid])

    return agg_kernel(x, ei4, zeros_feat, zeros_deg)


def _tc_combine_kernel(agg_ref, deg_ref, x_ref, wl_ref, bl_ref, wr_ref, o_ref):
    agg = agg_ref[0] + agg_ref[1]
    deg = jnp.sum(deg_ref[:, 0, :], axis=0, keepdims=True)
    recip = 1.0 / jnp.maximum(deg, 1.0)
    ones_row = jnp.ones((1, agg.shape[1]), jnp.float32)
    recip_col = lax.dot_general(recip, ones_row, (((0,), (0,)), ((), ())),
                                precision=lax.Precision.HIGHEST,
                                preferred_element_type=jnp.float32)
    mean = agg * recip_col
    out = (jnp.dot(mean, wl_ref[...], preferred_element_type=jnp.float32)
           + bl_ref[...]
           + jnp.dot(x_ref[...], wr_ref[...], preferred_element_type=jnp.float32))
    o_ref[...] = jnp.maximum(out, 0.0)


def _tc_combine(agg_p, deg_p, x_pad, W_l, b_l, W_r):
    Np, D = x_pad.shape
    NW = deg_p.shape[0]
    b2 = b_l.reshape(1, D)
    return pl.pallas_call(
        _tc_combine_kernel,
        grid=(1,),
        in_specs=[
            pl.BlockSpec((2, Np, D), lambda i: (0, 0, 0)),
            pl.BlockSpec((NW, 1, Np), lambda i: (0, 0, 0)),
            pl.BlockSpec((Np, D), lambda i: (0, 0)),
            pl.BlockSpec((D, D), lambda i: (0, 0)),
            pl.BlockSpec((1, D), lambda i: (0, 0)),
            pl.BlockSpec((D, D), lambda i: (0, 0)),
        ],
        out_specs=pl.BlockSpec((Np, D), lambda i: (0, 0)),
        out_shape=jax.ShapeDtypeStruct((Np, D), jnp.float32),
    )(agg_p, deg_p, x_pad, W_l, b2, W_r)


def kernel(x, edge_index, W_l, b_l, W_r):
    N, D = x.shape
    NS, NW = 16, 32
    n_per_tile = (N + NS * 8 - 1) // (NS * 8) * 8
    Np = n_per_tile * NS
    E = edge_index.shape[1]
    epw = E // NW
    assert epw * NW == E and N < Np
    C = 80
    nch = (epw + C - 1) // C
    pad = nch * C - epw
    if pad:
        ei3 = edge_index.reshape(2, NW, epw)
        srcp = jnp.pad(ei3[0], ((0, 0), (0, pad)))
        dpad = N + jnp.arange(pad, dtype=jnp.int32) % (Np - N)
        dstp = jnp.concatenate(
            [ei3[1], jnp.broadcast_to(dpad, (NW, pad))], axis=1)
        ei4 = jnp.stack([srcp, dstp]).reshape(
            2, NW, nch, C).transpose(1, 2, 0, 3)
    else:
        ei4 = edge_index.reshape(2, NW, nch, C).transpose(1, 2, 0, 3)
    zeros_feat = jnp.zeros((n_per_tile, D), jnp.float32)
    zeros_deg = jnp.zeros((1, Np), jnp.float32)
    agg_p, deg_p = _sc_aggregate(x, ei4, zeros_feat, zeros_deg,
                                 Np, n_per_tile)
    x_pad = jnp.pad(x, ((0, Np - N), (0, 0)))
    return _tc_combine(agg_p, deg_p, x_pad, W_l, b_l, W_r)[:N]


# speedup vs baseline: 1.5251x; 1.0490x over previous
import dataclasses
import functools

import jax
import jax.numpy as jnp
from jax import lax
from jax.experimental import pallas as pl
from jax.experimental.pallas import tpu as pltpu
from jax.experimental.pallas import tpu_sc as plsc


def _sc_aggregate(x, ei4, zeros_feat, zeros_deg, Np, n_per_tile):
    N, D = x.shape
    mesh = plsc.VectorSubcoreMesh(core_axis_name="c", subcore_axis_name="s")
    NC, NS = mesh.num_cores, mesh.num_subcores
    NW = NC * NS
    _, nch, _, C = ei4.shape
    assert nch >= 5
    main = ((nch - 1) // 4) * 4

    @functools.partial(
        pl.kernel,
        out_type=(
            jax.ShapeDtypeStruct((NC, Np, D), jnp.float32),
            jax.ShapeDtypeStruct((NW, 1, Np), jnp.float32),
        ),
        mesh=mesh,
        scratch_types=[
            pltpu.VMEM((2, C), jnp.int32),
            pltpu.VMEM((2, C), jnp.int32),
            pltpu.VMEM((2, C), jnp.int32),
            pltpu.VMEM((2, C), jnp.int32),
            pltpu.VMEM((C, D), jnp.float32),
            pltpu.VMEM((C, D), jnp.float32),
            pltpu.VMEM((1, Np), jnp.float32),
            pltpu.VMEM_SHARED((Np, D), jnp.float32),
            pltpu.SemaphoreType.DMA,
            pltpu.SemaphoreType.DMA,
            pltpu.SemaphoreType.DMA,
            pltpu.SemaphoreType.DMA,
            pltpu.SemaphoreType.DMA,
            pltpu.SemaphoreType.DMA,
            pltpu.SemaphoreType.DMA,
            pltpu.SemaphoreType.DMA,
        ],
        compiler_params=dataclasses.replace(pltpu.CompilerParams(),
                                            needs_layout_passes=False),
    )
    def agg_kernel(x_hbm, ei_hbm, zf_hbm, zd_hbm, agg_out, deg_out,
                   idx0, idx1, idx2, idx3, rows0, rows1, deg_v, agg_sh,
                   sem_i0, sem_i1, sem_i2, sem_i3,
                   sem_g0, sem_g1, sem_s0, sem_s1):
        core = lax.axis_index("c")
        sub = lax.axis_index("s")
        wid = core * NS + sub
        base_n = sub * n_per_tile
        idxb = (idx0, idx1, idx2, idx3)
        sem_i = (sem_i0, sem_i1, sem_i2, sem_i3)
        rows = (rows0, rows1)
        sem_g = (sem_g0, sem_g1)
        sem_s = (sem_s0, sem_s1)

        def idx_copy(c, b4):
            return pltpu.make_async_copy(ei_hbm.at[wid, c], idxb[b4],
                                         sem_i[b4])

        def gather(c, rb, b4):
            return pltpu.make_async_copy(x_hbm.at[idxb[b4].at[0]], rows[rb],
                                         sem_g[rb])

        def scatter_wait(rb, b4):
            pltpu.make_async_copy(rows[rb], agg_sh.at[idxb[b4].at[1]],
                                  sem_s[rb]).wait()

        idx_copy(0, 0).start()
        idx_copy(1, 1).start()
        idx_copy(2, 2).start()
        pltpu.sync_copy(zf_hbm, agg_sh.at[pl.ds(base_n, n_per_tile)])
        pltpu.sync_copy(zd_hbm, deg_v)
        idx_copy(0, 0).wait()
        gather(0, 0, 0).start()
        plsc.subcore_barrier()

        zero16 = jnp.zeros((16,), jnp.int32)
        one16 = jnp.ones((16,), jnp.float32)

        def hist(b4):
            for j in range(C // 16):
                idx = idxb[b4][1, pl.ds(j * 16, 16)]
                plsc.addupdate_scatter(deg_v, [zero16, idx], one16)

        @pl.loop(0, main, step=4)
        def _(i):
            for b in range(4):
                c = i + b
                rb = b & 1
                b4 = b & 3
                gather(c, rb, b4).wait()

                @pl.when(c >= 1)
                def _():
                    scatter_wait(rb ^ 1, (b4 - 1) % 4)

                idx_copy(c + 1, (b4 + 1) % 4).wait()
                gather(c + 1, rb ^ 1, (b4 + 1) % 4).start()

                @pl.when(c + 3 < nch)
                def _():
                    idx_copy(c + 3, (b4 + 3) % 4).start()

                pltpu.async_copy(rows[rb], agg_sh.at[idxb[b4].at[1]],
                                 sem_s[rb], add=True)
                hist(b4)

        for c in range(main, nch):
            rb, b4 = c & 1, c % 4
            gather(c, rb, b4).wait()
            if c >= 1:
                scatter_wait((c - 1) & 1, (c - 1) % 4)
            if c + 1 < nch:
                idx_copy(c + 1, (c + 1) % 4).wait()
                gather(c + 1, (c + 1) & 1, (c + 1) % 4).start()
            if c + 3 < nch:
                idx_copy(c + 3, (c + 3) % 4).start()
            pltpu.async_copy(rows[rb], agg_sh.at[idxb[b4].at[1]],
                             sem_s[rb], add=True)
            hist(b4)
        scatter_wait((nch - 1) & 1, (nch - 1) % 4)

        plsc.subcore_barrier()
        pltpu.sync_copy(agg_sh.at[pl.ds(base_n, n_per_tile)],
                        agg_out.at[core, pl.ds(base_n, n_per_tile)])
        pltpu.sync_copy(deg_v, deg_out.at[wid])

    return agg_kernel(x, ei4, zeros_feat, zeros_deg)


def _tc_combine_kernel(agg_ref, deg_ref, x_ref, wl_ref, bl_ref, wr_ref, o_ref):
    agg = agg_ref[0] + agg_ref[1]
    N, D = agg.shape
    deg = jnp.sum(deg_ref[:, 0, :], axis=0, keepdims=True)
    recip = 1.0 / jnp.maximum(deg, 1.0)
    ones_row = jnp.ones((1, D), jnp.float32)
    recip_col = lax.dot_general(recip, ones_row, (((0,), (0,)), ((), ())),
                                precision=lax.Precision.HIGHEST,
                                preferred_element_type=jnp.float32)
    mean = agg * lax.slice(recip_col, (0, 0), (N, D))
    out = (jnp.dot(mean, wl_ref[...], preferred_element_type=jnp.float32)
           + bl_ref[...]
           + jnp.dot(x_ref[...], wr_ref[...], preferred_element_type=jnp.float32))
    o_ref[...] = jnp.maximum(out, 0.0)


def _tc_combine(agg_p, deg_p, x, W_l, b_l, W_r):
    N, D = x.shape
    NW, _, Np = deg_p.shape
    b2 = b_l.reshape(1, D)
    return pl.pallas_call(
        _tc_combine_kernel,
        grid=(1,),
        in_specs=[
            pl.BlockSpec((2, N, D), lambda i: (0, 0, 0)),
            pl.BlockSpec((NW, 1, Np), lambda i: (0, 0, 0)),
            pl.BlockSpec((N, D), lambda i: (0, 0)),
            pl.BlockSpec((D, D), lambda i: (0, 0)),
            pl.BlockSpec((1, D), lambda i: (0, 0)),
            pl.BlockSpec((D, D), lambda i: (0, 0)),
        ],
        out_specs=pl.BlockSpec((N, D), lambda i: (0, 0)),
        out_shape=jax.ShapeDtypeStruct((N, D), jnp.float32),
    )(agg_p, deg_p, x, W_l, b2, W_r)


def kernel(x, edge_index, W_l, b_l, W_r):
    N, D = x.shape
    NS, NW = 16, 32
    n_per_tile = (N + NS * 8 - 1) // (NS * 8) * 8
    Np = n_per_tile * NS
    E = edge_index.shape[1]
    epw = E // NW
    assert epw * NW == E and N < Np
    C = 80
    nch = (epw + C - 1) // C
    pad = nch * C - epw
    if pad:
        ei3 = edge_index.reshape(2, NW, epw)
        srcp = jnp.pad(ei3[0], ((0, 0), (0, pad)))
        dpad = N + jnp.arange(pad, dtype=jnp.int32) % (Np - N)
        dstp = jnp.concatenate(
            [ei3[1], jnp.broadcast_to(dpad, (NW, pad))], axis=1)
        ei4 = jnp.stack([srcp, dstp]).reshape(
            2, NW, nch, C).transpose(1, 2, 0, 3)
    else:
        ei4 = edge_index.reshape(2, NW, nch, C).transpose(1, 2, 0, 3)
    zeros_feat = jnp.zeros((n_per_tile, D), jnp.float32)
    zeros_deg = jnp.zeros((1, Np), jnp.float32)
    agg_p, deg_p = _sc_aggregate(x, ei4, zeros_feat, zeros_deg,
                                 Np, n_per_tile)
    return _tc_combine(agg_p, deg_p, x, W_l, b_l, W_r)


# speedup vs baseline: 2.1061x; 1.3809x over previous
import dataclasses
import functools

import jax
import jax.numpy as jnp
from jax import lax
from jax.experimental import pallas as pl
from jax.experimental.pallas import tpu as pltpu
from jax.experimental.pallas import tpu_sc as plsc


def _sc_aggregate(x, ei4, zeros_feat, zeros_deg, Np, n_per_tile):
    N, D = x.shape
    mesh = plsc.VectorSubcoreMesh(core_axis_name="c", subcore_axis_name="s")
    NC, NS = mesh.num_cores, mesh.num_subcores
    NW = NC * NS
    _, nch, _, C = ei4.shape
    assert nch >= 5
    main = (nch // 12) * 12

    @functools.partial(
        pl.kernel,
        out_type=(
            jax.ShapeDtypeStruct((NC, Np, D), jnp.float32),
            jax.ShapeDtypeStruct((NW, 1, Np), jnp.float32),
        ),
        mesh=mesh,
        scratch_types=[
            pltpu.VMEM((2, C), jnp.int32),
            pltpu.VMEM((2, C), jnp.int32),
            pltpu.VMEM((2, C), jnp.int32),
            pltpu.VMEM((2, C), jnp.int32),
            pltpu.VMEM((C, D), jnp.float32),
            pltpu.VMEM((C, D), jnp.float32),
            pltpu.VMEM((C, D), jnp.float32),
            pltpu.VMEM((1, Np), jnp.float32),
            pltpu.VMEM_SHARED((Np, D), jnp.float32),
            pltpu.SemaphoreType.DMA,
            pltpu.SemaphoreType.DMA,
            pltpu.SemaphoreType.DMA,
            pltpu.SemaphoreType.DMA,
            pltpu.SemaphoreType.DMA,
            pltpu.SemaphoreType.DMA,
            pltpu.SemaphoreType.DMA,
            pltpu.SemaphoreType.DMA,
            pltpu.SemaphoreType.DMA,
            pltpu.SemaphoreType.DMA,
        ],
        compiler_params=dataclasses.replace(pltpu.CompilerParams(),
                                            needs_layout_passes=False),
    )
    def agg_kernel(x_hbm, ei_hbm, zf_hbm, zd_hbm, agg_out, deg_out,
                   idx0, idx1, idx2, idx3, rows0, rows1, rows2, deg_v, agg_sh,
                   sem_i0, sem_i1, sem_i2, sem_i3,
                   sem_g0, sem_g1, sem_g2, sem_s0, sem_s1, sem_s2):
        core = lax.axis_index("c")
        sub = lax.axis_index("s")
        wid = core * NS + sub
        base_n = sub * n_per_tile
        idxb = (idx0, idx1, idx2, idx3)
        sem_i = (sem_i0, sem_i1, sem_i2, sem_i3)
        rows = (rows0, rows1, rows2)
        sem_g = (sem_g0, sem_g1, sem_g2)
        sem_s = (sem_s0, sem_s1, sem_s2)

        def idx_copy(c, b4):
            return pltpu.make_async_copy(ei_hbm.at[wid, c], idxb[b4],
                                         sem_i[b4])

        def gather(c, rb, b4):
            return pltpu.make_async_copy(x_hbm.at[idxb[b4].at[0]], rows[rb],
                                         sem_g[rb])

        def scatter_wait(rb, b4):
            pltpu.make_async_copy(rows[rb], agg_sh.at[idxb[b4].at[1]],
                                  sem_s[rb]).wait()

        idx_copy(0, 0).start()
        idx_copy(1, 1).start()
        idx_copy(2, 2).start()
        pltpu.sync_copy(zf_hbm, agg_sh.at[pl.ds(base_n, n_per_tile)])
        pltpu.sync_copy(zd_hbm, deg_v)
        idx_copy(0, 0).wait()
        gather(0, 0, 0).start()
        idx_copy(1, 1).wait()
        gather(1, 1, 1).start()
        plsc.subcore_barrier()

        zero16 = jnp.zeros((16,), jnp.int32)
        one16 = jnp.ones((16,), jnp.float32)

        def hist(b4):
            for j in range(C // 16):
                idx = idxb[b4][1, pl.ds(j * 16, 16)]
                plsc.addupdate_scatter(deg_v, [zero16, idx], one16)

        @pl.loop(0, main, step=12)
        def _(i):
            for b in range(12):
                c = i + b
                r3 = b % 3
                b4 = b % 4
                gather(c, r3, b4).wait()

                @pl.when(c >= 1)
                def _():
                    scatter_wait((b - 1) % 3, (b - 1) % 4)

                @pl.when(c + 2 < nch)
                def _():
                    idx_copy(c + 2, (b4 + 2) % 4).wait()
                    gather(c + 2, (b + 2) % 3, (b4 + 2) % 4).start()

                @pl.when(c + 3 < nch)
                def _():
                    idx_copy(c + 3, (b4 + 3) % 4).start()

                pltpu.async_copy(rows[r3], agg_sh.at[idxb[b4].at[1]],
                                 sem_s[r3], add=True)
                hist(b4)

        for c in range(main, nch):
            r3, b4 = c % 3, c % 4
            gather(c, r3, b4).wait()
            if c >= 1:
                scatter_wait((c - 1) % 3, (c - 1) % 4)
            if c + 2 < nch:
                idx_copy(c + 2, (c + 2) % 4).wait()
                gather(c + 2, (c + 2) % 3, (c + 2) % 4).start()
            if c + 3 < nch:
                idx_copy(c + 3, (c + 3) % 4).start()
            pltpu.async_copy(rows[r3], agg_sh.at[idxb[b4].at[1]],
                             sem_s[r3], add=True)
            hist(b4)
        scatter_wait((nch - 1) % 3, (nch - 1) % 4)

        plsc.subcore_barrier()
        pltpu.sync_copy(agg_sh.at[pl.ds(base_n, n_per_tile)],
                        agg_out.at[core, pl.ds(base_n, n_per_tile)])
        pltpu.sync_copy(deg_v, deg_out.at[wid])

    return agg_kernel(x, ei4, zeros_feat, zeros_deg)


def _tc_combine_kernel(agg_ref, deg_ref, x_ref, wl_ref, bl_ref, wr_ref, o_ref):
    agg = agg_ref[0] + agg_ref[1]
    N, D = agg.shape
    deg = jnp.sum(deg_ref[:, 0, :], axis=0, keepdims=True)
    recip = 1.0 / jnp.maximum(deg, 1.0)
    ones_row = jnp.ones((1, D), jnp.float32)
    recip_col = lax.dot_general(recip, ones_row, (((0,), (0,)), ((), ())),
                                precision=lax.Precision.HIGHEST,
                                preferred_element_type=jnp.float32)
    mean = agg * lax.slice(recip_col, (0, 0), (N, D))
    out = (jnp.dot(mean, wl_ref[...], preferred_element_type=jnp.float32)
           + bl_ref[...]
           + jnp.dot(x_ref[...], wr_ref[...], preferred_element_type=jnp.float32))
    o_ref[...] = jnp.maximum(out, 0.0)


def _tc_combine(agg_p, deg_p, x, W_l, b_l, W_r):
    N, D = x.shape
    NW, _, Np = deg_p.shape
    b2 = b_l.reshape(1, D)
    return pl.pallas_call(
        _tc_combine_kernel,
        grid=(1,),
        in_specs=[
            pl.BlockSpec((2, N, D), lambda i: (0, 0, 0)),
            pl.BlockSpec((NW, 1, Np), lambda i: (0, 0, 0)),
            pl.BlockSpec((N, D), lambda i: (0, 0)),
            pl.BlockSpec((D, D), lambda i: (0, 0)),
            pl.BlockSpec((1, D), lambda i: (0, 0)),
            pl.BlockSpec((D, D), lambda i: (0, 0)),
        ],
        out_specs=pl.BlockSpec((N, D), lambda i: (0, 0)),
        out_shape=jax.ShapeDtypeStruct((N, D), jnp.float32),
    )(agg_p, deg_p, x, W_l, b2, W_r)


def kernel(x, edge_index, W_l, b_l, W_r):
    N, D = x.shape
    NS, NW = 16, 32
    n_per_tile = (N + NS * 8 - 1) // (NS * 8) * 8
    Np = n_per_tile * NS
    E = edge_index.shape[1]
    epw = E // NW
    assert epw * NW == E and N < Np
    C = 80
    nch = (epw + C - 1) // C
    pad = nch * C - epw
    if pad:
        ei3 = edge_index.reshape(2, NW, epw)
        srcp = jnp.pad(ei3[0], ((0, 0), (0, pad)))
        dpad = N + jnp.arange(pad, dtype=jnp.int32) % (Np - N)
        dstp = jnp.concatenate(
            [ei3[1], jnp.broadcast_to(dpad, (NW, pad))], axis=1)
        ei4 = jnp.stack([srcp, dstp]).reshape(
            2, NW, nch, C).transpose(1, 2, 0, 3)
    else:
        ei4 = edge_index.reshape(2, NW, nch, C).transpose(1, 2, 0, 3)
    zeros_feat = jnp.zeros((n_per_tile, D), jnp.float32)
    zeros_deg = jnp.zeros((1, Np), jnp.float32)
    agg_p, deg_p = _sc_aggregate(x, ei4, zeros_feat, zeros_deg,
                                 Np, n_per_tile)
    return _tc_combine(agg_p, deg_p, x, W_l, b_l, W_r)
